# baseline (speedup 1.0000x reference)
import jax
import jax.numpy as jnp
from jax import lax
from jax.experimental import pallas as pl
from jax.experimental.pallas import tpu as pltpu
from jax.experimental.pallas import tpu_sc as plsc

_VOCAB = 1000000
_EMBED = 64
_BATCH = 16384
_K = 5
_PAIR = 2 * _EMBED

_VBLK = 8192
_NBLK = 62
_SPLIT = _VBLK * _NBLK
_DOFF = 499712
_DBLK = _DOFF // _VBLK
_MSUB = 128
_NSUB = _VBLK // _MSUB

_NC = 2
_NS = 16
_NW = _NC * _NS
_BPW = _BATCH // _NW
_CHUNK = 64
_NCHUNK = _BPW // _CHUNK
_GRP = 16
_NGRP = _CHUNK // _GRP
_UNROLL = 4


def _transpose_body(eye_ref, xa_ref, xb_ref, y_ref):
  for k in range(_NSUB):
    sub = pl.ds(k * _MSUB, _MSUB)
    x = jnp.concatenate([xa_ref[:, sub], xb_ref[:, sub]], axis=0)
    y_ref[sub, :] = jax.lax.dot_general(
        eye_ref[...], x.astype(jnp.bfloat16),
        dimension_numbers=(((1,), (1,)), ((), ())),
        preferred_element_type=jnp.float32)


_transpose_call = pl.pallas_call(
    _transpose_body,
    grid=(_NBLK,),
    in_specs=[pl.BlockSpec((_MSUB, _MSUB), lambda j: (0, 0)),
              pl.BlockSpec((_EMBED, _VBLK), lambda j: (0, j)),
              pl.BlockSpec((_EMBED, _VBLK), lambda j: (0, j + _DBLK))],
    out_specs=pl.BlockSpec((_VBLK, _PAIR), lambda j: (j, 0)),
    out_shape=jax.ShapeDtypeStruct((_SPLIT, _PAIR), jnp.float32),
)


def _skipgram_body(tidx_hbm, cidx_hbm, ttab_hbm, ctab_hbm, out_hbm,
                   tidx_v, cidx_v, tslot_v, cslot_v, trows, crows, out_v,
                   sem0, sem1):
  wid = lax.axis_index("s") * _NC + lax.axis_index("c")

  pltpu.sync_copy(tidx_hbm.at[pl.ds(wid * _BPW, _BPW)], tidx_v)
  pltpu.sync_copy(cidx_hbm.at[pl.ds(wid * _BPW * _K, _BPW * _K)], cidx_v)

  split = jnp.full((_GRP,), _SPLIT, jnp.int32)
  doff = jnp.full((_GRP,), _DOFF, jnp.int32)
  for i in range(_BPW // _GRP):
    v = tidx_v[pl.ds(i * _GRP, _GRP)]
    tslot_v[pl.ds(i * _GRP, _GRP)] = jnp.where(v >= split, v - doff, v)
  for i in range(_BPW * _K // _GRP):
    v = cidx_v[pl.ds(i * _GRP, _GRP)]
    cslot_v[pl.ds(i * _GRP, _GRP)] = jnp.where(v >= split, v - doff, v)

  iota = lax.iota(jnp.int32, _GRP)
  off64 = jnp.full((_GRP,), _EMBED, jnp.int32)
  zero16 = jnp.zeros((_GRP,), jnp.int32)
  sems = (sem0, sem1)

  def issue(c):
    buf = c % 2
    sem = sems[buf]
    descs = [
        pltpu.async_copy(
            ttab_hbm.at[tslot_v.at[pl.ds(c * _CHUNK, _CHUNK)]],
            trows.at[pl.ds(buf * _CHUNK, _CHUNK)], sem)
    ]
    for r in range(_K):
      descs.append(
          pltpu.async_copy(
              ctab_hbm.at[cslot_v.at[pl.ds((c * _K + r) * _CHUNK, _CHUNK)]],
              crows.at[pl.ds((buf * _K + r) * _CHUNK, _CHUNK)], sem))
    return descs

  pending = issue(0)
  for c in range(_NCHUNK):
    nxt = issue(c + 1) if c + 1 < _NCHUNK else []
    for d in pending:
      d.wait()
    pending = nxt
    buf = c % 2

    def group(g, _):
      lane_b = g * _GRP + iota
      trow = buf * _CHUNK + lane_b
      crow0 = (buf * _K) * _CHUNK + lane_b * _K
      gbase = c * _CHUNK + lane_b
      tv = plsc.load_gather(tidx_v, [gbase])
      tcol0 = jnp.where(tv >= split, off64, zero16)
      ccol0 = [
          jnp.where(
              plsc.load_gather(cidx_v, [gbase * _K + j]) >= split,
              off64, zero16)
          for j in range(_K)
      ]

      def body(_, carry):
        tcol = carry[0]
        ccols = carry[1:1 + _K]
        accs = carry[1 + _K:]
        for _u in range(_UNROLL):
          we = plsc.load_gather(trows, [trow, tcol])
          accs = tuple(
              accs[j] +
              plsc.load_gather(crows, [crow0 + j, ccols[j]]) * we
              for j in range(_K))
          tcol = tcol + 1
          ccols = tuple(cc + 1 for cc in ccols)
        return (tcol,) + ccols + accs

      zero = jnp.zeros((_GRP,), jnp.float32)
      init = (tcol0,) + tuple(ccol0) + (zero,) * _K
      res = lax.fori_loop(0, _EMBED // _UNROLL, body, init)
      accs = res[1 + _K:]

      obase = gbase * _K
      for j in range(_K):
        plsc.store_scatter(out_v, [obase + j], accs[j])
      return 0

    lax.fori_loop(0, _NGRP, group, 0)

  pltpu.sync_copy(out_v, out_hbm.at[pl.ds(wid * _BPW * _K, _BPW * _K)])


@jax.jit
def _skipgram(tidx, cidx, ttabT, ctabT):
  eye = jnp.eye(_MSUB, dtype=jnp.bfloat16)
  ttab = _transpose_call(eye, ttabT, ttabT)
  ctab = _transpose_call(eye, ctabT, ctabT)
  mesh = plsc.VectorSubcoreMesh(core_axis_name="c", subcore_axis_name="s",
                                num_cores=_NC, num_subcores=_NS)
  kern = pl.kernel(
      _skipgram_body,
      out_type=jax.ShapeDtypeStruct((_BATCH * _K,), jnp.float32),
      mesh=mesh,
      compiler_params=pltpu.CompilerParams(needs_layout_passes=False),
      scratch_types=[
          pltpu.VMEM((_BPW,), jnp.int32),
          pltpu.VMEM((_BPW * _K,), jnp.int32),
          pltpu.VMEM((_BPW,), jnp.int32),
          pltpu.VMEM((_BPW * _K,), jnp.int32),
          pltpu.VMEM((2 * _CHUNK, _PAIR), jnp.float32),
          pltpu.VMEM((2 * _CHUNK * _K, _PAIR), jnp.float32),
          pltpu.VMEM((_BPW * _K,), jnp.float32),
          pltpu.SemaphoreType.DMA,
          pltpu.SemaphoreType.DMA,
      ],
  )
  return kern(tidx, cidx, ttab, ctab)


def kernel(target, context, target_table, context_table):
  tidx = target.reshape(_BATCH)
  cidx = context.reshape(_BATCH * _K)
  out = _skipgram(tidx, cidx, target_table.T, context_table.T)
  return out.reshape(_BATCH, _K)


# speedup vs baseline: 1.0475x; 1.0475x over previous
import jax
import jax.numpy as jnp
from jax import lax
from jax.experimental import pallas as pl
from jax.experimental.pallas import tpu as pltpu
from jax.experimental.pallas import tpu_sc as plsc

_VOCAB = 1000000
_EMBED = 64
_BATCH = 16384
_K = 5
_PAIR = 2 * _EMBED

_VBLK = 16384
_NBLK = 31
_SPLIT = _VBLK * _NBLK
_DOFF = 507904
_DBLK = _DOFF // _VBLK
_MSUB = 256
_NSUB = _VBLK // _MSUB

_NC = 2
_NS = 16
_NW = _NC * _NS
_BPW = _BATCH // _NW
_CHUNK = 64
_NCHUNK = _BPW // _CHUNK
_GRP = 16
_NGRP = _CHUNK // _GRP
_UNROLL = 4


def _transpose_body(eye_ref, xa_ref, xb_ref, y_ref):
  for k in range(_NSUB):
    sub = pl.ds(k * _MSUB, _MSUB)
    x = jnp.concatenate([xa_ref[:, sub], xb_ref[:, sub]], axis=0)
    y_ref[sub, :] = jax.lax.dot_general(
        eye_ref[...], x.astype(jnp.bfloat16),
        dimension_numbers=(((1,), (1,)), ((), ())),
        preferred_element_type=jnp.float32)


_transpose_call = pl.pallas_call(
    _transpose_body,
    grid=(_NBLK,),
    in_specs=[pl.BlockSpec((_MSUB, _MSUB), lambda j: (0, 0)),
              pl.BlockSpec((_EMBED, _VBLK), lambda j: (0, j)),
              pl.BlockSpec((_EMBED, _VBLK), lambda j: (0, j + _DBLK))],
    out_specs=pl.BlockSpec((_VBLK, _PAIR), lambda j: (j, 0)),
    out_shape=jax.ShapeDtypeStruct((_SPLIT, _PAIR), jnp.float32),
)


def _skipgram_body(tidx_hbm, cidx_hbm, ttab_hbm, ctab_hbm, out_hbm,
                   tidx_v, cidx_v, tslot_v, cslot_v, trows, crows, out_v,
                   sem0, sem1):
  wid = lax.axis_index("s") * _NC + lax.axis_index("c")

  pltpu.sync_copy(tidx_hbm.at[pl.ds(wid * _BPW, _BPW)], tidx_v)
  pltpu.sync_copy(cidx_hbm.at[pl.ds(wid * _BPW * _K, _BPW * _K)], cidx_v)

  split = jnp.full((_GRP,), _SPLIT, jnp.int32)
  doff = jnp.full((_GRP,), _DOFF, jnp.int32)
  for i in range(_BPW // _GRP):
    v = tidx_v[pl.ds(i * _GRP, _GRP)]
    tslot_v[pl.ds(i * _GRP, _GRP)] = jnp.where(v >= split, v - doff, v)
  for i in range(_BPW * _K // _GRP):
    v = cidx_v[pl.ds(i * _GRP, _GRP)]
    cslot_v[pl.ds(i * _GRP, _GRP)] = jnp.where(v >= split, v - doff, v)

  iota = lax.iota(jnp.int32, _GRP)
  off64 = jnp.full((_GRP,), _EMBED, jnp.int32)
  zero16 = jnp.zeros((_GRP,), jnp.int32)
  sems = (sem0, sem1)

  def issue(c):
    buf = c % 2
    sem = sems[buf]
    descs = [
        pltpu.async_copy(
            ttab_hbm.at[tslot_v.at[pl.ds(c * _CHUNK, _CHUNK)]],
            trows.at[pl.ds(buf * _CHUNK, _CHUNK)], sem)
    ]
    for r in range(_K):
      descs.append(
          pltpu.async_copy(
              ctab_hbm.at[cslot_v.at[pl.ds((c * _K + r) * _CHUNK, _CHUNK)]],
              crows.at[pl.ds((buf * _K + r) * _CHUNK, _CHUNK)], sem))
    return descs

  pending = issue(0)
  for c in range(_NCHUNK):
    nxt = issue(c + 1) if c + 1 < _NCHUNK else []
    for d in pending:
      d.wait()
    pending = nxt
    buf = c % 2

    def group(g, _):
      lane_b = g * _GRP + iota
      trow = buf * _CHUNK + lane_b
      crow0 = (buf * _K) * _CHUNK + lane_b * _K
      gbase = c * _CHUNK + lane_b
      tv = plsc.load_gather(tidx_v, [gbase])
      tcol0 = jnp.where(tv >= split, off64, zero16)
      ccol0 = [
          jnp.where(
              plsc.load_gather(cidx_v, [gbase * _K + j]) >= split,
              off64, zero16)
          for j in range(_K)
      ]

      def body(_, carry):
        tcol = carry[0]
        ccols = carry[1:1 + _K]
        accs = carry[1 + _K:]
        for _u in range(_UNROLL):
          we = plsc.load_gather(trows, [trow, tcol])
          accs = tuple(
              accs[j] +
              plsc.load_gather(crows, [crow0 + j, ccols[j]]) * we
              for j in range(_K))
          tcol = tcol + 1
          ccols = tuple(cc + 1 for cc in ccols)
        return (tcol,) + ccols + accs

      zero = jnp.zeros((_GRP,), jnp.float32)
      init = (tcol0,) + tuple(ccol0) + (zero,) * _K
      res = lax.fori_loop(0, _EMBED // _UNROLL, body, init)
      accs = res[1 + _K:]

      obase = gbase * _K
      for j in range(_K):
        plsc.store_scatter(out_v, [obase + j], accs[j])
      return 0

    lax.fori_loop(0, _NGRP, group, 0)

  pltpu.sync_copy(out_v, out_hbm.at[pl.ds(wid * _BPW * _K, _BPW * _K)])


@jax.jit
def _skipgram(tidx, cidx, ttabT, ctabT):
  eye = jnp.eye(_MSUB, dtype=jnp.bfloat16)
  ttab = _transpose_call(eye, ttabT, ttabT)
  ctab = _transpose_call(eye, ctabT, ctabT)
  mesh = plsc.VectorSubcoreMesh(core_axis_name="c", subcore_axis_name="s",
                                num_cores=_NC, num_subcores=_NS)
  kern = pl.kernel(
      _skipgram_body,
      out_type=jax.ShapeDtypeStruct((_BATCH * _K,), jnp.float32),
      mesh=mesh,
      compiler_params=pltpu.CompilerParams(needs_layout_passes=False),
      scratch_types=[
          pltpu.VMEM((_BPW,), jnp.int32),
          pltpu.VMEM((_BPW * _K,), jnp.int32),
          pltpu.VMEM((_BPW,), jnp.int32),
          pltpu.VMEM((_BPW * _K,), jnp.int32),
          pltpu.VMEM((2 * _CHUNK, _PAIR), jnp.float32),
          pltpu.VMEM((2 * _CHUNK * _K, _PAIR), jnp.float32),
          pltpu.VMEM((_BPW * _K,), jnp.float32),
          pltpu.SemaphoreType.DMA,
          pltpu.SemaphoreType.DMA,
      ],
  )
  return kern(tidx, cidx, ttab, ctab)


def kernel(target, context, target_table, context_table):
  tidx = target.reshape(_BATCH)
  cidx = context.reshape(_BATCH * _K)
  out = _skipgram(tidx, cidx, target_table.T, context_table.T)
  return out.reshape(_BATCH, _K)
